# baseline (speedup 1.0000x reference)
import jax
import jax.numpy as jnp
from jax.experimental import pallas as pl

_N = 10000
_NEG = 0.2
_V = 3


def _gat(x, ei, W, a_s, a_d, b):
    h = x @ W.T
    sl = jnp.arange(_N, dtype=ei.dtype)
    src = jnp.concatenate([ei[0], sl])
    dst = jnp.concatenate([ei[1], sl])
    a_src = (h * a_s).sum(-1)
    a_dst = (h * a_d).sum(-1)
    e = a_src[src] + a_dst[dst]
    e = jnp.where(e >= 0.0, e, _NEG * e)
    m = jax.ops.segment_max(e, dst, num_segments=_N)
    ex = jnp.exp(e - m[dst])
    s = jax.ops.segment_sum(ex, dst, num_segments=_N)
    alpha = ex / (s[dst] + 1e-16)
    out = jax.ops.segment_sum(h[src] * alpha[:, None], dst, num_segments=_N)
    return out + b


def _recon_body(zi_ref, zj_ref, out_ref):
    acc = jnp.dot(zi_ref[...], zj_ref[...].T, preferred_element_type=jnp.float32)
    out_ref[...] = jax.nn.sigmoid(acc)


def _recon(zn):
    BM = BN = 512
    grid = (pl.cdiv(_N, BM), pl.cdiv(_N, BN))
    return pl.pallas_call(
        _recon_body,
        grid=grid,
        in_specs=[
            pl.BlockSpec((BM, 128), lambda i, j: (i, 0)),
            pl.BlockSpec((BN, 128), lambda i, j: (j, 0)),
        ],
        out_specs=pl.BlockSpec((BM, BN), lambda i, j: (i, j)),
        out_shape=jax.ShapeDtypeStruct((_N, _N), jnp.float32),
    )(zn, zn)


def kernel(x, edge_index_v0, edge_index_v1, edge_index_v2,
           W1_0, as1_0, ad1_0, b1_0, W2_0, as2_0, ad2_0, b2_0,
           W1_1, as1_1, ad1_1, b1_1, W2_1, as2_1, ad2_1, b2_1,
           W1_2, as1_2, ad1_2, b1_2, W2_2, as2_2, ad2_2, b2_2,
           Wv1, bv1, Wv2, bv2):
    eis = [edge_index_v0, edge_index_v1, edge_index_v2]
    params1 = [(W1_0, as1_0, ad1_0, b1_0), (W1_1, as1_1, ad1_1, b1_1),
               (W1_2, as1_2, ad1_2, b1_2)]
    params2 = [(W2_0, as2_0, ad2_0, b2_0), (W2_1, as2_1, ad2_1, b2_1),
               (W2_2, as2_2, ad2_2, b2_2)]
    outs = []
    for v in range(_V):
        h = _gat(x, eis[v], *params1[v])
        h = jax.nn.elu(h)
        h = _gat(h, eis[v], *params2[v])
        outs.append(h)
    stacked = jnp.stack(outs)
    hidden = jnp.maximum(jnp.einsum('vnd,kd->vnk', stacked, Wv1) + bv1, 0.0)
    scores = (jnp.einsum('vnk,ok->vno', hidden, Wv2) + bv2)[:, :, 0]
    w = jax.nn.softmax(scores.T, axis=1)
    z = (stacked * w.T[:, :, None]).sum(0)
    z_relu = jnp.maximum(z, 0.0)
    norm = jnp.sqrt(jnp.sum(z_relu * z_relu, axis=1, keepdims=True))
    z_norm = z_relu / jnp.maximum(norm, 1e-07)
    recon = _recon(z_norm)
    return (z, recon, w)


# speedup vs baseline: 4.1784x; 4.1784x over previous
import functools

import jax
import jax.numpy as jnp
from jax import lax
from jax.experimental import pallas as pl
from jax.experimental.pallas import tpu as pltpu
from jax.experimental.pallas import tpu_sc as plsc

_N = 10000
_E = 320000
_NEG = 0.2
_V = 3
_D = 128

_NP = 10240
_EP = 331776
_NBT = 162
_NR = _NP // 128
_SL = _NP // 16
_CH = 2560


def _pre_body(x_ref, W_ref, as_ref, ad_ref, h_ref, asrc_ref, adst_ref):
    W = W_ref[0]
    h = lax.dot_general(x_ref[...], W, (((1,), (1,)), ((), ())),
                        preferred_element_type=jnp.float32)
    h_ref[0] = h
    asrc_ref[0, 0] = jnp.reshape(jnp.sum(h * as_ref[0, 0][None, :], axis=1),
                                 (1, 128))
    adst_ref[0, 0] = jnp.reshape(jnp.sum(h * ad_ref[0, 0][None, :], axis=1),
                                 (1, 128))


def _pre(x_pad, Ws, ass, ads):
    grid = (_V, _NR)
    return pl.pallas_call(
        _pre_body,
        grid=grid,
        in_specs=[
            pl.BlockSpec((128, _D), lambda v, n: (n, 0)),
            pl.BlockSpec((1, _D, _D), lambda v, n: (v, 0, 0)),
            pl.BlockSpec((1, 1, _D), lambda v, n: (v, 0, 0)),
            pl.BlockSpec((1, 1, _D), lambda v, n: (v, 0, 0)),
        ],
        out_specs=[
            pl.BlockSpec((1, 128, _D), lambda v, n: (v, n, 0)),
            pl.BlockSpec((1, 1, 1, 128), lambda v, n: (v, n, 0, 0)),
            pl.BlockSpec((1, 1, 1, 128), lambda v, n: (v, n, 0, 0)),
        ],
        out_shape=[
            jax.ShapeDtypeStruct((_V, _NP, _D), jnp.float32),
            jax.ShapeDtypeStruct((_V, _NR, 1, 128), jnp.float32),
            jax.ShapeDtypeStruct((_V, _NR, 1, 128), jnp.float32),
        ],
    )(x_pad, Ws, ass.reshape(_V, 1, _D), ads.reshape(_V, 1, _D))


def _mid_body(agg_ref, den_ref, b_ref, W_ref, as_ref, ad_ref,
              h_ref, asrc_ref, adst_ref):
    p = agg_ref[0]
    s = jnp.sum(den_ref[0], axis=0)
    o = p / (s + 1e-16)
    xin = o + b_ref[0, 0][None, :]
    x2 = jnp.where(xin > 0, xin, jnp.exp(jnp.minimum(xin, 0.0)) - 1.0)
    W = W_ref[0]
    h = lax.dot_general(x2, W, (((1,), (1,)), ((), ())),
                        preferred_element_type=jnp.float32)
    h_ref[0] = h
    asrc_ref[0, 0] = jnp.reshape(jnp.sum(h * as_ref[0, 0][None, :], axis=1),
                                 (1, 128))
    adst_ref[0, 0] = jnp.reshape(jnp.sum(h * ad_ref[0, 0][None, :], axis=1),
                                 (1, 128))


def _mid(agg, den, bs, Ws, ass, ads):
    grid = (_V, _NR)
    return pl.pallas_call(
        _mid_body,
        grid=grid,
        in_specs=[
            pl.BlockSpec((1, 128, _D), lambda v, n: (v, n, 0)),
            pl.BlockSpec((1, 16, 128, 1), lambda v, n: (v, 0, n, 0)),
            pl.BlockSpec((1, 1, _D), lambda v, n: (v, 0, 0)),
            pl.BlockSpec((1, _D, _D), lambda v, n: (v, 0, 0)),
            pl.BlockSpec((1, 1, _D), lambda v, n: (v, 0, 0)),
            pl.BlockSpec((1, 1, _D), lambda v, n: (v, 0, 0)),
        ],
        out_specs=[
            pl.BlockSpec((1, 128, _D), lambda v, n: (v, n, 0)),
            pl.BlockSpec((1, 1, 1, 128), lambda v, n: (v, n, 0, 0)),
            pl.BlockSpec((1, 1, 1, 128), lambda v, n: (v, n, 0, 0)),
        ],
        out_shape=[
            jax.ShapeDtypeStruct((_V, _NP, _D), jnp.float32),
            jax.ShapeDtypeStruct((_V, _NR, 1, 128), jnp.float32),
            jax.ShapeDtypeStruct((_V, _NR, 1, 128), jnp.float32),
        ],
    )(agg, den.reshape(_V, 16, _NP, 1), bs.reshape(_V, 1, _D), Ws,
      ass.reshape(_V, 1, _D), ads.reshape(_V, 1, _D))


def _post_body(agg_ref, den_ref, b_ref, Wv1_ref, bv1_ref, Wv2_ref, bv2_ref,
               z_ref, zn_ref, w_ref):
    os = []
    scs = []
    for v in range(_V):
        p = agg_ref[v]
        s = jnp.sum(den_ref[v], axis=0)
        o = p / (s + 1e-16) + b_ref[v][None, :]
        os.append(o)
        hid = lax.dot_general(o, Wv1_ref[...], (((1,), (1,)), ((), ())),
                              preferred_element_type=jnp.float32)
        hid = jnp.maximum(hid + bv1_ref[0][None, :], 0.0)
        sc = jnp.sum(hid * Wv2_ref[0][None, :], axis=1, keepdims=True)
        scs.append(sc + bv2_ref[0, 0])
    m = jnp.maximum(jnp.maximum(scs[0], scs[1]), scs[2])
    es = [jnp.exp(s - m) for s in scs]
    tot = es[0] + es[1] + es[2]
    ws = [e / tot for e in es]
    z = ws[0] * os[0] + ws[1] * os[1] + ws[2] * os[2]
    z_ref[...] = z
    zr = jnp.maximum(z, 0.0)
    nrm = jnp.sqrt(jnp.sum(zr * zr, axis=1, keepdims=True))
    zn_ref[...] = zr / jnp.maximum(nrm, 1e-7)
    w_ref[...] = jnp.concatenate(
        [jnp.reshape(w, (1, 1, 1, 128)) for w in ws], axis=0)


def _post(agg, den, bs, Wv1, bv1, Wv2, bv2):
    grid = (_NR,)
    return pl.pallas_call(
        _post_body,
        grid=grid,
        in_specs=[
            pl.BlockSpec((_V, 128, _D), lambda n: (0, n, 0)),
            pl.BlockSpec((_V, 16, 128, 1), lambda n: (0, 0, n, 0)),
            pl.BlockSpec((_V, _D), lambda n: (0, 0)),
            pl.BlockSpec((32, _D), lambda n: (0, 0)),
            pl.BlockSpec((1, 32), lambda n: (0, 0)),
            pl.BlockSpec((1, 32), lambda n: (0, 0)),
            pl.BlockSpec((1, 1), lambda n: (0, 0)),
        ],
        out_specs=[
            pl.BlockSpec((128, _D), lambda n: (n, 0)),
            pl.BlockSpec((128, _D), lambda n: (n, 0)),
            pl.BlockSpec((_V, 1, 1, 128), lambda n: (0, n, 0, 0)),
        ],
        out_shape=[
            jax.ShapeDtypeStruct((_NP, _D), jnp.float32),
            jax.ShapeDtypeStruct((_NP, _D), jnp.float32),
            jax.ShapeDtypeStruct((_V, _NR, 1, 128), jnp.float32),
        ],
    )(agg, den.reshape(_V, 16, _NP, 1), bs, Wv1, bv1.reshape(1, 32), Wv2,
      bv2.reshape(1, 1))


def _recon_body(zi_ref, zj_ref, out_ref):
    acc = lax.dot_general(zi_ref[...], zj_ref[...], (((1,), (1,)), ((), ())),
                          preferred_element_type=jnp.float32)
    out_ref[...] = jax.nn.sigmoid(acc)


def _recon(zn):
    BM = BN = 512
    grid = (pl.cdiv(_N, BM), pl.cdiv(_N, BN))
    return pl.pallas_call(
        _recon_body,
        grid=grid,
        in_specs=[
            pl.BlockSpec((BM, _D), lambda i, j: (i, 0)),
            pl.BlockSpec((BN, _D), lambda i, j: (j, 0)),
        ],
        out_specs=pl.BlockSpec((BM, BN), lambda i, j: (i, j)),
        out_shape=jax.ShapeDtypeStruct((_N, _N), jnp.float32),
    )(zn, zn)


def _sc_body(h_hbm, asrc_h, adst_h, srcm, dstm, agg_out, den_out,
             asrc_v, adst_v, src_v, dst_v, dloc_v, rows_v, zero_v, sloc_v,
             acc_sh, sem):
    s = lax.axis_index("s")
    wid = s

    zv = jnp.zeros((16,), jnp.float32)
    for r in range(64):
        for q in range(8):
            zero_v[r, pl.ds(q * 16, 16)] = zv

    lane = lax.broadcasted_iota(jnp.int32, (16,), 0)
    masks = [lane == r for r in range(16)]

    @pl.loop(0, _V)
    def _view(v):
        pltpu.sync_copy(asrc_h.at[pl.ds(v * _NP, _NP)], asrc_v)
        pltpu.sync_copy(adst_h.at[pl.ds(v * _NP, _NP)], adst_v)
        pltpu.sync_copy(srcm.at[v, wid], src_v)
        pltpu.sync_copy(dstm.at[v, wid], dst_v)

        @pl.loop(0, _NP // 16)
        def _zs(i):
            sloc_v[pl.ds(i * 16, 16)] = zv

        @pl.loop(0, 4)
        def _chunk(b):
            lo = b * _CH
            for t in range(2):
                pltpu.sync_copy(
                    zero_v, acc_sh.at[pl.ds(s * (_CH // 16) + t * 64, 64)])
            pltpu.sync_copy(
                zero_v.at[pl.ds(0, 32)],
                acc_sh.at[pl.ds(s * (_CH // 16) + 128, 32)])
            plsc.subcore_barrier()

            @pl.loop(0, _NBT)
            def _edge_batch(j):
                cp = pltpu.async_copy(h_hbm.at[src_v.at[j]], rows_v, sem)
                for k in range(8):
                    dv = dst_v[j, pl.ds(k * 16, 16)]
                    dloc_v[0, pl.ds(k * 16, 16)] = jnp.clip(dv - lo, 0, _CH - 1)
                cp.wait()
                for k in range(8):
                    sv = src_v[j, pl.ds(k * 16, 16)]
                    dv = dst_v[j, pl.ds(k * 16, 16)]
                    a1 = plsc.load_gather(asrc_v, [sv - v * _NP])
                    a2 = plsc.load_gather(adst_v, [dv])
                    e = a1 + a2
                    e = jnp.where(e >= 0.0, e, _NEG * e)
                    ex = jnp.exp(e)

                    @pl.when(b == 0)
                    def _den():
                        for r in range(16):
                            plsc.addupdate_scatter(sloc_v, [dv], ex,
                                                   mask=masks[r])

                    inb = (dv >= lo) & (dv < lo + _CH)
                    exm = jnp.where(inb, ex, 0.0)
                    for r in range(16):
                        a_r = exm[r]
                        rr = k * 16 + r
                        for q in range(8):
                            rows_v[rr, pl.ds(q * 16, 16)] = (
                                rows_v[rr, pl.ds(q * 16, 16)] * a_r)
                pltpu.sync_copy(rows_v, acc_sh.at[dloc_v.at[0]], add=True)

            plsc.subcore_barrier()
            pltpu.sync_copy(
                acc_sh.at[pl.ds(s * (_CH // 16), _CH // 16)],
                agg_out.at[pl.ds(v * _NP + lo + s * (_CH // 16),
                                 _CH // 16)])
            plsc.subcore_barrier()

        pltpu.sync_copy(sloc_v, den_out.at[pl.ds((v * 16 + s) * _NP, _NP)])


def _sc_agg(h_flat, asrc_flat, adst_flat, srcm, dstm):
    mesh = plsc.VectorSubcoreMesh(core_axis_name="c", subcore_axis_name="s",
                                  num_cores=1)
    f = functools.partial(
        pl.kernel,
        out_type=[
            jax.ShapeDtypeStruct((_V * _NP, _D), jnp.float32),
            jax.ShapeDtypeStruct((_V * 16 * _NP,), jnp.float32),
        ],
        mesh=mesh,
        compiler_params=pltpu.CompilerParams(needs_layout_passes=False),
        scratch_types=[
            pltpu.VMEM((_NP,), jnp.float32),
            pltpu.VMEM((_NP,), jnp.float32),
            pltpu.VMEM((_NBT, 128), jnp.int32),
            pltpu.VMEM((_NBT, 128), jnp.int32),
            pltpu.VMEM((1, 128), jnp.int32),
            pltpu.VMEM((128, _D), jnp.float32),
            pltpu.VMEM((64, _D), jnp.float32),
            pltpu.VMEM((_NP,), jnp.float32),
            pltpu.VMEM_SHARED((_CH, _D), jnp.float32),
            pltpu.SemaphoreType.DMA,
        ],
    )(_sc_body)
    return f(h_flat, asrc_flat, adst_flat, srcm, dstm)


def kernel(x, edge_index_v0, edge_index_v1, edge_index_v2,
           W1_0, as1_0, ad1_0, b1_0, W2_0, as2_0, ad2_0, b2_0,
           W1_1, as1_1, ad1_1, b1_1, W2_1, as2_1, ad2_1, b2_1,
           W1_2, as1_2, ad1_2, b1_2, W2_2, as2_2, ad2_2, b2_2,
           Wv1, bv1, Wv2, bv2):
    eis = [edge_index_v0, edge_index_v1, edge_index_v2]
    W1s = jnp.stack([W1_0, W1_1, W1_2])
    as1s = jnp.stack([as1_0, as1_1, as1_2])
    ad1s = jnp.stack([ad1_0, ad1_1, ad1_2])
    b1s = jnp.stack([b1_0, b1_1, b1_2])
    W2s = jnp.stack([W2_0, W2_1, W2_2])
    as2s = jnp.stack([as2_0, as2_1, as2_2])
    ad2s = jnp.stack([ad2_0, ad2_1, ad2_2])
    b2s = jnp.stack([b2_0, b2_1, b2_2])

    x_pad = jnp.pad(x, ((0, _NP - _N), (0, 0)))

    sl = jnp.arange(_N, dtype=jnp.int32)
    pad = jnp.full((_EP - _E - _N,), _N, dtype=jnp.int32)
    srcs, dsts = [], []
    for v in range(_V):
        srcs.append(jnp.concatenate([eis[v][0], sl, pad]) + v * _NP)
        dsts.append(jnp.concatenate([eis[v][1], sl, pad]))
    srcm = jnp.stack(srcs).reshape(_V, 16, _NBT, 128)
    dstm = jnp.stack(dsts).reshape(_V, 16, _NBT, 128)

    h1, asrc1, adst1 = _pre(x_pad, W1s, as1s, ad1s)
    agg1, den1 = _sc_agg(h1.reshape(_V * _NP, _D), asrc1.reshape(_V * _NP),
                         adst1.reshape(_V * _NP), srcm, dstm)
    agg1 = agg1.reshape(_V, _NP, _D)
    h2, asrc2, adst2 = _mid(agg1, den1, b1s, W2s, as2s, ad2s)
    agg2, den2 = _sc_agg(h2.reshape(_V * _NP, _D), asrc2.reshape(_V * _NP),
                         adst2.reshape(_V * _NP), srcm, dstm)
    agg2 = agg2.reshape(_V, _NP, _D)
    z, zn, w_t = _post(agg2, den2, b2s, Wv1, bv1, Wv2, bv2)
    recon = _recon(zn[:_N])
    w = w_t.reshape(_V, _NP).T[:_N]
    return (z[:_N], recon, w)


# speedup vs baseline: 15.0519x; 3.6023x over previous
import functools

import jax
import jax.numpy as jnp
from jax import lax
from jax.experimental import pallas as pl
from jax.experimental.pallas import tpu as pltpu
from jax.experimental.pallas import tpu_sc as plsc

_N = 10000
_E = 320000
_NEG = 0.2
_V = 3
_D = 128

_NP = 10240
_EP = 331776
_NBT = 162
_NR = _NP // 128
_SL = _NP // 16
_CH = 1792


def _pre_body(x_ref, W_ref, as_ref, ad_ref, h_ref, asrc_ref, adst_ref):
    W = W_ref[0]
    h = lax.dot_general(x_ref[...], W, (((1,), (1,)), ((), ())),
                        preferred_element_type=jnp.float32)
    h_ref[0] = h
    asrc_ref[0, 0] = jnp.reshape(jnp.sum(h * as_ref[0, 0][None, :], axis=1),
                                 (1, 128))
    adst_ref[0, 0] = jnp.reshape(jnp.sum(h * ad_ref[0, 0][None, :], axis=1),
                                 (1, 128))


def _pre(x_pad, Ws, ass, ads):
    grid = (_V, _NR)
    return pl.pallas_call(
        _pre_body,
        grid=grid,
        in_specs=[
            pl.BlockSpec((128, _D), lambda v, n: (n, 0)),
            pl.BlockSpec((1, _D, _D), lambda v, n: (v, 0, 0)),
            pl.BlockSpec((1, 1, _D), lambda v, n: (v, 0, 0)),
            pl.BlockSpec((1, 1, _D), lambda v, n: (v, 0, 0)),
        ],
        out_specs=[
            pl.BlockSpec((1, 128, _D), lambda v, n: (v, n, 0)),
            pl.BlockSpec((1, 1, 1, 128), lambda v, n: (v, n, 0, 0)),
            pl.BlockSpec((1, 1, 1, 128), lambda v, n: (v, n, 0, 0)),
        ],
        out_shape=[
            jax.ShapeDtypeStruct((_V, _NP, _D), jnp.float32),
            jax.ShapeDtypeStruct((_V, _NR, 1, 128), jnp.float32),
            jax.ShapeDtypeStruct((_V, _NR, 1, 128), jnp.float32),
        ],
    )(x_pad, Ws, ass.reshape(_V, 1, _D), ads.reshape(_V, 1, _D))


def _mid_body(agg_ref, den_ref, b_ref, W_ref, as_ref, ad_ref,
              h_ref, asrc_ref, adst_ref):
    p = agg_ref[0]
    s = den_ref[0]
    o = p / (s + 1e-16)
    xin = o + b_ref[0, 0][None, :]
    x2 = jnp.where(xin > 0, xin, jnp.exp(jnp.minimum(xin, 0.0)) - 1.0)
    W = W_ref[0]
    h = lax.dot_general(x2, W, (((1,), (1,)), ((), ())),
                        preferred_element_type=jnp.float32)
    h_ref[0] = h
    asrc_ref[0, 0] = jnp.reshape(jnp.sum(h * as_ref[0, 0][None, :], axis=1),
                                 (1, 128))
    adst_ref[0, 0] = jnp.reshape(jnp.sum(h * ad_ref[0, 0][None, :], axis=1),
                                 (1, 128))


def _mid(agg, den, bs, Ws, ass, ads):
    grid = (_V, _NR)
    return pl.pallas_call(
        _mid_body,
        grid=grid,
        in_specs=[
            pl.BlockSpec((1, 128, _D), lambda v, n: (v, n, 0)),
            pl.BlockSpec((1, 128, 1), lambda v, n: (v, n, 0)),
            pl.BlockSpec((1, 1, _D), lambda v, n: (v, 0, 0)),
            pl.BlockSpec((1, _D, _D), lambda v, n: (v, 0, 0)),
            pl.BlockSpec((1, 1, _D), lambda v, n: (v, 0, 0)),
            pl.BlockSpec((1, 1, _D), lambda v, n: (v, 0, 0)),
        ],
        out_specs=[
            pl.BlockSpec((1, 128, _D), lambda v, n: (v, n, 0)),
            pl.BlockSpec((1, 1, 1, 128), lambda v, n: (v, n, 0, 0)),
            pl.BlockSpec((1, 1, 1, 128), lambda v, n: (v, n, 0, 0)),
        ],
        out_shape=[
            jax.ShapeDtypeStruct((_V, _NP, _D), jnp.float32),
            jax.ShapeDtypeStruct((_V, _NR, 1, 128), jnp.float32),
            jax.ShapeDtypeStruct((_V, _NR, 1, 128), jnp.float32),
        ],
    )(agg, den.reshape(_V, _NP, 1), bs.reshape(_V, 1, _D), Ws,
      ass.reshape(_V, 1, _D), ads.reshape(_V, 1, _D))


def _post_body(agg_ref, den_ref, b_ref, Wv1_ref, bv1_ref, Wv2_ref, bv2_ref,
               z_ref, zn_ref, w_ref):
    os = []
    scs = []
    for v in range(_V):
        p = agg_ref[v]
        s = den_ref[v]
        o = p / (s + 1e-16) + b_ref[v][None, :]
        os.append(o)
        hid = lax.dot_general(o, Wv1_ref[...], (((1,), (1,)), ((), ())),
                              preferred_element_type=jnp.float32)
        hid = jnp.maximum(hid + bv1_ref[0][None, :], 0.0)
        sc = jnp.sum(hid * Wv2_ref[0][None, :], axis=1, keepdims=True)
        scs.append(sc + bv2_ref[0, 0])
    m = jnp.maximum(jnp.maximum(scs[0], scs[1]), scs[2])
    es = [jnp.exp(s - m) for s in scs]
    tot = es[0] + es[1] + es[2]
    ws = [e / tot for e in es]
    z = ws[0] * os[0] + ws[1] * os[1] + ws[2] * os[2]
    z_ref[...] = z
    zr = jnp.maximum(z, 0.0)
    nrm = jnp.sqrt(jnp.sum(zr * zr, axis=1, keepdims=True))
    zn_ref[...] = zr / jnp.maximum(nrm, 1e-7)
    w_ref[...] = jnp.concatenate(
        [jnp.reshape(w, (1, 1, 1, 128)) for w in ws], axis=0)


def _post(agg, den, bs, Wv1, bv1, Wv2, bv2):
    grid = (_NR,)
    return pl.pallas_call(
        _post_body,
        grid=grid,
        in_specs=[
            pl.BlockSpec((_V, 128, _D), lambda n: (0, n, 0)),
            pl.BlockSpec((_V, 128, 1), lambda n: (0, n, 0)),
            pl.BlockSpec((_V, _D), lambda n: (0, 0)),
            pl.BlockSpec((32, _D), lambda n: (0, 0)),
            pl.BlockSpec((1, 32), lambda n: (0, 0)),
            pl.BlockSpec((1, 32), lambda n: (0, 0)),
            pl.BlockSpec((1, 1), lambda n: (0, 0)),
        ],
        out_specs=[
            pl.BlockSpec((128, _D), lambda n: (n, 0)),
            pl.BlockSpec((128, _D), lambda n: (n, 0)),
            pl.BlockSpec((_V, 1, 1, 128), lambda n: (0, n, 0, 0)),
        ],
        out_shape=[
            jax.ShapeDtypeStruct((_NP, _D), jnp.float32),
            jax.ShapeDtypeStruct((_NP, _D), jnp.float32),
            jax.ShapeDtypeStruct((_V, _NR, 1, 128), jnp.float32),
        ],
    )(agg, den.reshape(_V, _NP, 1), bs, Wv1, bv1.reshape(1, 32), Wv2,
      bv2.reshape(1, 1))


def _denred_body(in_ref, out_ref):
    out_ref[0] = jnp.sum(in_ref[0], axis=0, keepdims=True)


def _denred(den):
    return pl.pallas_call(
        _denred_body,
        grid=(_V,),
        in_specs=[pl.BlockSpec((1, 32, _NP), lambda v: (v, 0, 0))],
        out_specs=pl.BlockSpec((1, 1, _NP), lambda v: (v, 0, 0)),
        out_shape=jax.ShapeDtypeStruct((_V, 1, _NP), jnp.float32),
    )(den.reshape(_V, 32, _NP))


def _recon_body(zi_ref, zj_ref, out_ref):
    acc = lax.dot_general(zi_ref[...], zj_ref[...], (((1,), (1,)), ((), ())),
                          preferred_element_type=jnp.float32)
    out_ref[...] = jax.nn.sigmoid(acc)


def _recon(zn):
    BM = BN = 512
    grid = (pl.cdiv(_N, BM), pl.cdiv(_N, BN))
    return pl.pallas_call(
        _recon_body,
        grid=grid,
        in_specs=[
            pl.BlockSpec((BM, _D), lambda i, j: (i, 0)),
            pl.BlockSpec((BN, _D), lambda i, j: (j, 0)),
        ],
        out_specs=pl.BlockSpec((BM, BN), lambda i, j: (i, j)),
        out_shape=jax.ShapeDtypeStruct((_N, _N), jnp.float32),
    )(zn, zn)


def _sc_body(h_hbm, asrc_h, adst_h, pkm, agg_out, den_out,
             asrc_v, adst_v, opk_v, cpk_v, srow_v, dloc_v, rows_v, zero_v,
             sloc_v, acc_sh, sem):
    c = lax.axis_index("c")
    s = lax.axis_index("s")
    wid = s

    zv = jnp.zeros((16,), jnp.float32)
    for r in range(64):
        for q in range(8):
            zero_v[r, pl.ds(q * 16, 16)] = zv

    lane = lax.broadcasted_iota(jnp.int32, (16,), 0)
    masks = [lane == r for r in range(16)]

    @pl.loop(0, _V)
    def _view(v):
        pltpu.sync_copy(asrc_h.at[pl.ds(v * _NP, _NP)], asrc_v)
        pltpu.sync_copy(adst_h.at[pl.ds(v * _NP, _NP)], adst_v)
        pltpu.sync_copy(pkm.at[v, wid], opk_v)

        @pl.loop(0, _NP // 16)
        def _zs(i):
            sloc_v[pl.ds(i * 16, 16)] = zv

        @pl.loop(0, 3)
        def _chunk(b):
            lo = (c * 3 + b) * _CH
            pltpu.sync_copy(zero_v, acc_sh.at[pl.ds(s * (_CH // 16), 64)])
            pltpu.sync_copy(
                zero_v.at[pl.ds(0, 48)],
                acc_sh.at[pl.ds(s * (_CH // 16) + 64, 48)])

            def _cbody(g, off):
                row = g // 8
                k = g % 8
                pk = opk_v[row, pl.ds(k * 16, 16)]
                dv = lax.bitwise_and(pk, 16383)
                m = (dv >= lo) & (dv < lo + _CH)
                plsc.store_compressed(cpk_v.at[pl.ds(off, 16)], pk, mask=m)
                return off + plsc.all_reduce_population_count(m)[0]

            off = lax.fori_loop(0, _NBT * 8, _cbody, jnp.int32(0))
            pzv = jnp.zeros((16,), jnp.int32) + ((v * _NP + _NP - 1) * 16384
                                                 + lo)
            for i in range(8):
                cpk_v[pl.ds(off + i * 16, 16)] = pzv
            nb = (off + 127) // 128
            plsc.subcore_barrier()

            @pl.loop(0, nb)
            def _edge_batch(j):
                for k in range(8):
                    pk = cpk_v[pl.ds(j * 128 + k * 16, 16)]
                    srow_v[0, pl.ds(k * 16, 16)] = (
                        lax.shift_right_logical(pk, 14))
                cp = pltpu.async_copy(h_hbm.at[srow_v.at[0]], rows_v, sem)
                exs = []
                for k in range(8):
                    pk = cpk_v[pl.ds(j * 128 + k * 16, 16)]
                    sv = lax.shift_right_logical(pk, 14)
                    dv = lax.bitwise_and(pk, 16383)
                    dloc_v[0, pl.ds(k * 16, 16)] = jnp.clip(dv - lo, 0,
                                                            _CH - 1)
                    a1 = plsc.load_gather(asrc_v, [sv - v * _NP])
                    a2 = plsc.load_gather(adst_v, [dv])
                    e = a1 + a2
                    e = jnp.where(e >= 0.0, e, _NEG * e)
                    ex = jnp.exp(e)
                    exs.append(ex)
                    for r in range(16):
                        plsc.addupdate_scatter(sloc_v, [dv], ex,
                                               mask=masks[r])
                cp.wait()
                for k in range(8):
                    ex = exs[k]
                    for r in range(16):
                        a_r = ex[r]
                        rr = k * 16 + r
                        for q in range(8):
                            rows_v[rr, pl.ds(q * 16, 16)] = (
                                rows_v[rr, pl.ds(q * 16, 16)] * a_r)
                pltpu.sync_copy(rows_v, acc_sh.at[dloc_v.at[0]], add=True)

            plsc.subcore_barrier()
            pltpu.sync_copy(
                acc_sh.at[pl.ds(s * (_CH // 16), _CH // 16)],
                agg_out.at[pl.ds(v * 6 * _CH + lo + s * (_CH // 16),
                                 _CH // 16)])
            plsc.subcore_barrier()

        pltpu.sync_copy(
            sloc_v,
            den_out.at[pl.ds(((v * 2 + c) * 16 + s) * _NP, _NP)])


def _sc_agg(h_flat, asrc_flat, adst_flat, pkm):
    mesh = plsc.VectorSubcoreMesh(core_axis_name="c", subcore_axis_name="s")
    f = functools.partial(
        pl.kernel,
        out_type=[
            jax.ShapeDtypeStruct((_V * 6 * _CH, _D), jnp.float32),
            jax.ShapeDtypeStruct((_V * 32 * _NP,), jnp.float32),
        ],
        mesh=mesh,
        compiler_params=pltpu.CompilerParams(needs_layout_passes=False),
        scratch_types=[
            pltpu.VMEM((_NP,), jnp.float32),
            pltpu.VMEM((_NP,), jnp.float32),
            pltpu.VMEM((_NBT, 128), jnp.int32),
            pltpu.VMEM((_NBT * 128 + 128,), jnp.int32),
            pltpu.VMEM((1, 128), jnp.int32),
            pltpu.VMEM((1, 128), jnp.int32),
            pltpu.VMEM((128, _D), jnp.float32),
            pltpu.VMEM((64, _D), jnp.float32),
            pltpu.VMEM((_NP,), jnp.float32),
            pltpu.VMEM_SHARED((_CH, _D), jnp.float32),
            pltpu.SemaphoreType.DMA,
        ],
    )(_sc_body)
    return f(h_flat, asrc_flat, adst_flat, pkm)


def kernel(x, edge_index_v0, edge_index_v1, edge_index_v2,
           W1_0, as1_0, ad1_0, b1_0, W2_0, as2_0, ad2_0, b2_0,
           W1_1, as1_1, ad1_1, b1_1, W2_1, as2_1, ad2_1, b2_1,
           W1_2, as1_2, ad1_2, b1_2, W2_2, as2_2, ad2_2, b2_2,
           Wv1, bv1, Wv2, bv2):
    eis = [edge_index_v0, edge_index_v1, edge_index_v2]
    W1s = jnp.stack([W1_0, W1_1, W1_2])
    as1s = jnp.stack([as1_0, as1_1, as1_2])
    ad1s = jnp.stack([ad1_0, ad1_1, ad1_2])
    b1s = jnp.stack([b1_0, b1_1, b1_2])
    W2s = jnp.stack([W2_0, W2_1, W2_2])
    as2s = jnp.stack([as2_0, as2_1, as2_2])
    ad2s = jnp.stack([ad2_0, ad2_1, ad2_2])
    b2s = jnp.stack([b2_0, b2_1, b2_2])

    x_pad = jnp.pad(x, ((0, _NP - _N), (0, 0)))

    sl = jnp.arange(_N, dtype=jnp.int32)
    pad_s = jnp.full((_EP - _E - _N,), _NP - 1, dtype=jnp.int32)
    pad_d = jnp.arange(_EP - _E - _N, dtype=jnp.int32) * 5 % _N
    pks = []
    for v in range(_V):
        sgv = jnp.concatenate([eis[v][0], sl, pad_s]) + v * _NP
        dgv = jnp.concatenate([eis[v][1], sl, pad_d])
        pks.append(sgv * 16384 + dgv)
    pkm = jnp.stack(pks).reshape(_V, 16, _NBT, 128)

    poison_rows = jnp.array([_NP - 1, 2 * _NP - 1, 3 * _NP - 1])

    h1, asrc1, adst1 = _pre(x_pad, W1s, as1s, ad1s)
    asrc1f = asrc1.reshape(_V * _NP).at[poison_rows].set(-1e30)
    agg1, den1 = _sc_agg(h1.reshape(_V * _NP, _D), asrc1f,
                         adst1.reshape(_V * _NP), pkm)
    agg1 = agg1.reshape(_V, 6 * _CH, _D)[:, :_NP]
    den1 = _denred(den1)
    h2, asrc2, adst2 = _mid(agg1, den1, b1s, W2s, as2s, ad2s)
    asrc2f = asrc2.reshape(_V * _NP).at[poison_rows].set(-1e30)
    agg2, den2 = _sc_agg(h2.reshape(_V * _NP, _D), asrc2f,
                         adst2.reshape(_V * _NP), pkm)
    agg2 = agg2.reshape(_V, 6 * _CH, _D)[:, :_NP]
    den2 = _denred(den2)
    z, zn, w_t = _post(agg2, den2, b2s, Wv1, bv1, Wv2, bv2)
    recon = _recon(zn[:_N])
    w = w_t.reshape(_V, _NP).T[:_N]
    return (z[:_N], recon, w)
